# baseline (speedup 1.0000x reference)
import functools

import jax
import jax.numpy as jnp
from jax import lax
from jax.experimental import pallas as pl
from jax.experimental.pallas import tpu as pltpu
from jax.experimental.pallas import tpu_sc as plsc

_B = 1024
_NCLASS = 1000
_NSUPP = 64
_NCHUNK = 4096
_NCLS_PAD = 1024

_NC = 2
_NS = 16
_NW = _NC * _NS
_CLS_PER_W = 32
_CLS_PER_ROUND = 8
_ROUNDS = _CLS_PER_W // _CLS_PER_ROUND
_LAST_CLS = _NCLASS - _CLS_PER_W


def _sc_build_w(chunk_map, wSupp):
    mesh = plsc.VectorSubcoreMesh(core_axis_name="c", subcore_axis_name="s")

    @functools.partial(
        pl.kernel,
        mesh=mesh,
        compiler_params=pltpu.CompilerParams(needs_layout_passes=False),
        out_type=jax.ShapeDtypeStruct((_NCLS_PAD, _NCHUNK), jnp.float32),
        scratch_types=[
            pltpu.VMEM((_CLS_PER_W, _NSUPP), jnp.int32),
            pltpu.VMEM((_CLS_PER_W, _NSUPP), jnp.float32),
            pltpu.VMEM((_CLS_PER_ROUND, _NCHUNK), jnp.float32),
            pltpu.VMEM((_CLS_PER_ROUND, _NCHUNK), jnp.float32),
            pltpu.SemaphoreType.DMA,
            pltpu.SemaphoreType.DMA,
            pltpu.SemaphoreType.DMA,
        ],
    )
    def k(cm_hbm, w_hbm, wt_hbm, cm_v, w_v, buf0, buf1, sem0, sem1, sem_in):
        wid = lax.axis_index("s") * _NC + lax.axis_index("c")
        base_cls = pl.multiple_of(jnp.minimum(wid * _CLS_PER_W, _LAST_CLS), 8)
        in_cm = pltpu.async_copy(
            cm_hbm.at[pl.ds(base_cls, _CLS_PER_W), :], cm_v, sem_in)
        in_w = pltpu.async_copy(
            w_hbm.at[pl.ds(base_cls, _CLS_PER_W), :], w_v, sem_in)

        zv = jnp.zeros((16,), jnp.float32)
        lane = lax.broadcasted_iota(jnp.int32, (16,), 0)
        lane8 = jnp.bitwise_and(lane, 7)
        lmask = lane < 8
        bufs = (buf0, buf1)
        sems = (sem0, sem1)
        copies = [None, None]

        def zero(buf):
            for row in range(_CLS_PER_ROUND):
                def zero_body(i, carry, row=row):
                    for u in range(8):
                        buf[row, pl.ds((i * 8 + u) * 16, 16)] = zv
                    return carry

                lax.fori_loop(0, _NCHUNK // (16 * 8), zero_body, 0)

        def scatter(buf, r):
            row_idx = r * _CLS_PER_ROUND + lane8
            for j in range(_NSUPP):
                col_j = jnp.full((16,), j, jnp.int32)
                cm_j = plsc.load_gather(cm_v, [row_idx, col_j], mask=lmask)
                w_j = plsc.load_gather(w_v, [row_idx, col_j], mask=lmask)
                plsc.addupdate_scatter(buf, [lane8, cm_j], w_j, mask=lmask)

        def unscatter(buf, r_prev):
            row_idx = r_prev * _CLS_PER_ROUND + lane8
            for j in range(_NSUPP):
                col_j = jnp.full((16,), j, jnp.int32)
                cm_j = plsc.load_gather(cm_v, [row_idx, col_j], mask=lmask)
                plsc.store_scatter(buf, [lane8, cm_j], zv, mask=lmask)

        zero(buf0)
        in_cm.wait()
        in_w.wait()
        for r in range(_ROUNDS):
            b = r & 1
            buf = bufs[b]
            if r == 1:
                zero(buf1)
            elif r >= 2:
                copies[b].wait()
                unscatter(buf, r - 2)
            scatter(buf, r)
            row0 = pl.multiple_of(base_cls + r * _CLS_PER_ROUND,
                                  _CLS_PER_ROUND)
            copies[b] = pltpu.async_copy(
                buf, wt_hbm.at[pl.ds(row0, _CLS_PER_ROUND), :], sems[b])
        copies[0].wait()
        copies[1].wait()

    return k(chunk_map, wSupp)


_BN = 128


def _tc_matmul_exp_t(xb, wt):
    def body(wt_ref, x_ref, o_ref):
        acc = lax.dot_general(
            wt_ref[...].astype(jnp.bfloat16), x_ref[...],
            (((1,), (1,)), ((), ())),
            preferred_element_type=jnp.float32)
        o_ref[...] = jnp.exp(acc)

    return pl.pallas_call(
        body,
        grid=(_NCLS_PAD // _BN,),
        in_specs=[
            pl.BlockSpec((_BN, _NCHUNK), lambda j: (j, 0)),
            pl.BlockSpec((_B, _NCHUNK), lambda j: (0, 0)),
        ],
        out_specs=pl.BlockSpec((_BN, _B), lambda j: (j, 0)),
        out_shape=jax.ShapeDtypeStruct((_NCLASS, _B), jnp.float32),
    )(wt, xb)


def kernel(x, wSupp, chunk_map):
    wt = _sc_build_w(chunk_map, wSupp)
    xb = x.astype(jnp.bfloat16)
    return _tc_matmul_exp_t(xb, wt).T


# speedup vs baseline: 1.2114x; 1.2114x over previous
import functools

import jax
import jax.numpy as jnp
from jax import lax
from jax.experimental import pallas as pl
from jax.experimental.pallas import tpu as pltpu
from jax.experimental.pallas import tpu_sc as plsc

_B = 1024
_NCLASS = 1000
_NSUPP = 64
_NCHUNK = 4096
_NCLS_PAD = 1024

_NC = 2
_NS = 16
_NW = _NC * _NS
_CLS_PER_W = 32
_CLS_PER_ROUND = 8
_ROUNDS = _CLS_PER_W // _CLS_PER_ROUND
_LAST_CLS = _NCLASS - _CLS_PER_W


def _sc_build_w(chunk_map, wSupp):
    mesh = plsc.VectorSubcoreMesh(core_axis_name="c", subcore_axis_name="s")

    @functools.partial(
        pl.kernel,
        mesh=mesh,
        compiler_params=pltpu.CompilerParams(needs_layout_passes=False),
        out_type=jax.ShapeDtypeStruct((_NCLS_PAD, _NCHUNK), jnp.float32),
        scratch_types=[
            pltpu.VMEM((_CLS_PER_W, _NSUPP), jnp.int32),
            pltpu.VMEM((_CLS_PER_W, _NSUPP), jnp.float32),
            pltpu.VMEM((_CLS_PER_ROUND, _NCHUNK), jnp.float32),
            pltpu.VMEM((_CLS_PER_ROUND, _NCHUNK), jnp.float32),
            pltpu.SemaphoreType.DMA,
            pltpu.SemaphoreType.DMA,
            pltpu.SemaphoreType.DMA,
        ],
    )
    def k(cm_hbm, w_hbm, wt_hbm, cm_v, w_v, buf0, buf1, sem0, sem1, sem_in):
        wid = lax.axis_index("s") * _NC + lax.axis_index("c")
        base_cls = pl.multiple_of(jnp.minimum(wid * _CLS_PER_W, _LAST_CLS), 8)
        in_cm = pltpu.async_copy(
            cm_hbm.at[pl.ds(base_cls, _CLS_PER_W), :], cm_v, sem_in)
        in_w = pltpu.async_copy(
            w_hbm.at[pl.ds(base_cls, _CLS_PER_W), :], w_v, sem_in)

        zv = jnp.zeros((16,), jnp.float32)
        lane = lax.broadcasted_iota(jnp.int32, (16,), 0)
        lane8 = jnp.bitwise_and(lane, 7)
        lmask = lane < 8
        bufs = (buf0, buf1)
        sems = (sem0, sem1)
        copies = [None, None]

        def zero(buf):
            for row in range(_CLS_PER_ROUND):
                def zero_body(i, carry, row=row):
                    for u in range(8):
                        buf[row, pl.ds((i * 8 + u) * 16, 16)] = zv
                    return carry

                lax.fori_loop(0, _NCHUNK // (16 * 8), zero_body, 0)

        def scatter(buf, r):
            row_idx = r * _CLS_PER_ROUND + lane8
            for j in range(_NSUPP):
                col_j = jnp.full((16,), j, jnp.int32)
                cm_j = plsc.load_gather(cm_v, [row_idx, col_j], mask=lmask)
                w_j = plsc.load_gather(w_v, [row_idx, col_j], mask=lmask)
                plsc.addupdate_scatter(buf, [lane8, cm_j], w_j, mask=lmask)

        def unscatter(buf, r_prev):
            row_idx = r_prev * _CLS_PER_ROUND + lane8
            for j in range(_NSUPP):
                col_j = jnp.full((16,), j, jnp.int32)
                cm_j = plsc.load_gather(cm_v, [row_idx, col_j], mask=lmask)
                plsc.store_scatter(buf, [lane8, cm_j], zv, mask=lmask)

        zero(buf0)
        in_cm.wait()
        in_w.wait()
        for r in range(_ROUNDS):
            b = r & 1
            buf = bufs[b]
            if r == 1:
                zero(buf1)
            elif r >= 2:
                copies[b].wait()
                unscatter(buf, r - 2)
            scatter(buf, r)
            row0 = pl.multiple_of(base_cls + r * _CLS_PER_ROUND,
                                  _CLS_PER_ROUND)
            copies[b] = pltpu.async_copy(
                buf, wt_hbm.at[pl.ds(row0, _CLS_PER_ROUND), :], sems[b])
        copies[0].wait()
        copies[1].wait()

    return k(chunk_map, wSupp)


_BN = 512


def _tc_matmul_exp_t(xb, wt):
    def body(wt_ref, x_ref, o_ref):
        acc = lax.dot_general(
            wt_ref[...].astype(jnp.bfloat16), x_ref[...],
            (((1,), (1,)), ((), ())),
            preferred_element_type=jnp.float32)
        o_ref[...] = jnp.exp(acc)

    return pl.pallas_call(
        body,
        grid=(_NCLS_PAD // _BN,),
        in_specs=[
            pl.BlockSpec((_BN, _NCHUNK), lambda j: (j, 0)),
            pl.BlockSpec((_B, _NCHUNK), lambda j: (0, 0)),
        ],
        out_specs=pl.BlockSpec((_BN, _B), lambda j: (j, 0)),
        out_shape=jax.ShapeDtypeStruct((_NCLASS, _B), jnp.float32),
    )(wt, xb)


def kernel(x, wSupp, chunk_map):
    wt = _sc_build_w(chunk_map, wSupp)
    xb = x.astype(jnp.bfloat16)
    return _tc_matmul_exp_t(xb, wt).T


# speedup vs baseline: 1.2331x; 1.0179x over previous
import functools

import jax
import jax.numpy as jnp
from jax import lax
from jax.experimental import pallas as pl
from jax.experimental.pallas import tpu as pltpu
from jax.experimental.pallas import tpu_sc as plsc

_B = 1024
_NCLASS = 1000
_NSUPP = 64
_NCHUNK = 4096
_NCLS_PAD = 1024

_NC = 2
_NS = 16
_NW = _NC * _NS
_CLS_PER_W = 32
_CLS_PER_ROUND = 8
_ROUNDS = _CLS_PER_W // _CLS_PER_ROUND
_LAST_CLS = _NCLASS - _CLS_PER_W


def _sc_build_w(chunk_map, wSupp):
    mesh = plsc.VectorSubcoreMesh(core_axis_name="c", subcore_axis_name="s")

    @functools.partial(
        pl.kernel,
        mesh=mesh,
        compiler_params=pltpu.CompilerParams(needs_layout_passes=False),
        out_type=jax.ShapeDtypeStruct((_NCLS_PAD, _NCHUNK), jnp.float32),
        scratch_types=[
            pltpu.VMEM((_CLS_PER_W, _NSUPP), jnp.int32),
            pltpu.VMEM((_CLS_PER_W, _NSUPP), jnp.float32),
            pltpu.VMEM((_CLS_PER_ROUND, _NCHUNK), jnp.float32),
            pltpu.VMEM((_CLS_PER_ROUND, _NCHUNK), jnp.float32),
            pltpu.SemaphoreType.DMA,
            pltpu.SemaphoreType.DMA,
            pltpu.SemaphoreType.DMA,
        ],
    )
    def k(cm_hbm, w_hbm, wt_hbm, cm_v, w_v, buf0, buf1, sem0, sem1, sem_in):
        wid = lax.axis_index("s") * _NC + lax.axis_index("c")
        base_cls = pl.multiple_of(jnp.minimum(wid * _CLS_PER_W, _LAST_CLS), 8)
        in_cm = pltpu.async_copy(
            cm_hbm.at[pl.ds(base_cls, _CLS_PER_W), :], cm_v, sem_in)
        in_w = pltpu.async_copy(
            w_hbm.at[pl.ds(base_cls, _CLS_PER_W), :], w_v, sem_in)

        zv = jnp.zeros((16,), jnp.float32)
        lane = lax.broadcasted_iota(jnp.int32, (16,), 0)
        lane8 = jnp.bitwise_and(lane, _CLS_PER_ROUND - 1)
        lmask = lane < _CLS_PER_ROUND
        bufs = (buf0, buf1)
        sems = (sem0, sem1)
        copies = [None, None]

        def zero(buf):
            for row in range(_CLS_PER_ROUND):
                def zero_body(i, carry, row=row):
                    for u in range(8):
                        buf[row, pl.ds((i * 8 + u) * 16, 16)] = zv
                    return carry

                lax.fori_loop(0, _NCHUNK // (16 * 8), zero_body, 0)

        def scatter(buf, r):
            row_idx = r * _CLS_PER_ROUND + lane8
            for j in range(_NSUPP):
                col_j = jnp.full((16,), j, jnp.int32)
                cm_j = plsc.load_gather(cm_v, [row_idx, col_j], mask=lmask)
                w_j = plsc.load_gather(w_v, [row_idx, col_j], mask=lmask)
                plsc.addupdate_scatter(buf, [lane8, cm_j], w_j, mask=lmask)

        def unscatter(buf, r_prev):
            row_idx = r_prev * _CLS_PER_ROUND + lane8
            for j in range(_NSUPP):
                col_j = jnp.full((16,), j, jnp.int32)
                cm_j = plsc.load_gather(cm_v, [row_idx, col_j], mask=lmask)
                plsc.store_scatter(buf, [lane8, cm_j], zv, mask=lmask)

        zero(buf0)
        in_cm.wait()
        in_w.wait()
        for r in range(_ROUNDS):
            b = r & 1
            buf = bufs[b]
            if r == 1:
                zero(buf1)
            elif r >= 2:
                copies[b].wait()
                unscatter(buf, r - 2)
            scatter(buf, r)
            row0 = pl.multiple_of(base_cls + r * _CLS_PER_ROUND,
                                  _CLS_PER_ROUND)
            copies[b] = pltpu.async_copy(
                buf, wt_hbm.at[pl.ds(row0, _CLS_PER_ROUND), :], sems[b])
        copies[0].wait()
        copies[1].wait()

    return k(chunk_map, wSupp)


_BN = 256


def _tc_matmul_exp_t(xb, wt):
    def body(wt_ref, x_ref, o_ref):
        acc = lax.dot_general(
            wt_ref[...].astype(jnp.bfloat16), x_ref[...],
            (((1,), (1,)), ((), ())),
            preferred_element_type=jnp.float32)
        o_ref[...] = jnp.exp(acc)

    return pl.pallas_call(
        body,
        grid=(_NCLS_PAD // _BN,),
        in_specs=[
            pl.BlockSpec((_BN, _NCHUNK), lambda j: (j, 0)),
            pl.BlockSpec((_B, _NCHUNK), lambda j: (0, 0)),
        ],
        out_specs=pl.BlockSpec((_BN, _B), lambda j: (j, 0)),
        out_shape=jax.ShapeDtypeStruct((_NCLASS, _B), jnp.float32),
    )(wt, xb)


def kernel(x, wSupp, chunk_map):
    wt = _sc_build_w(chunk_map, wSupp)
    xb = x.astype(jnp.bfloat16)
    return _tc_matmul_exp_t(xb, wt).T


# speedup vs baseline: 1.2370x; 1.0032x over previous
import functools

import jax
import jax.numpy as jnp
from jax import lax
from jax.experimental import pallas as pl
from jax.experimental.pallas import tpu as pltpu
from jax.experimental.pallas import tpu_sc as plsc

_B = 1024
_NCLASS = 1000
_NSUPP = 64
_NCHUNK = 4096
_NCLS_PAD = 1024

_NC = 2
_NS = 16
_NW = _NC * _NS
_CLS_PER_W = 32
_CLS_PER_ROUND = 8
_ROUNDS = _CLS_PER_W // _CLS_PER_ROUND
_ROWS2 = _NCLS_PAD // 2


def _sc_build_w(cm2, w2):
    mesh = plsc.VectorSubcoreMesh(core_axis_name="c", subcore_axis_name="s")

    @functools.partial(
        pl.kernel,
        mesh=mesh,
        compiler_params=pltpu.CompilerParams(needs_layout_passes=False),
        out_type=jax.ShapeDtypeStruct((_NCLS_PAD, _NCHUNK), jnp.float32),
        scratch_types=[
            pltpu.VMEM((_CLS_PER_W // 2, 2 * _NSUPP), jnp.int32),
            pltpu.VMEM((_CLS_PER_W // 2, 2 * _NSUPP), jnp.float32),
            pltpu.VMEM((_CLS_PER_ROUND, _NCHUNK), jnp.float32),
            pltpu.VMEM((_CLS_PER_ROUND, _NCHUNK), jnp.float32),
            pltpu.SemaphoreType.DMA,
            pltpu.SemaphoreType.DMA,
            pltpu.SemaphoreType.DMA,
        ],
    )
    def k(cm_hbm, w_hbm, wt_hbm, cm_v, w_v, buf0, buf1, sem0, sem1, sem_in):
        wid = lax.axis_index("s") * _NC + lax.axis_index("c")
        base_cls = pl.multiple_of(wid * _CLS_PER_W, 8)
        base_row2 = pl.multiple_of(wid * (_CLS_PER_W // 2), 8)
        in_cm = pltpu.async_copy(
            cm_hbm.at[pl.ds(base_row2, _CLS_PER_W // 2), :], cm_v, sem_in)
        in_w = pltpu.async_copy(
            w_hbm.at[pl.ds(base_row2, _CLS_PER_W // 2), :], w_v, sem_in)

        zv = jnp.zeros((16,), jnp.float32)
        lane = lax.broadcasted_iota(jnp.int32, (16,), 0)
        lane8 = jnp.bitwise_and(lane, _CLS_PER_ROUND - 1)
        lmask = lane < _CLS_PER_ROUND
        bufs = (buf0, buf1)
        sems = (sem0, sem1)
        copies = [None, None]

        def zero(buf):
            for row in range(_CLS_PER_ROUND):
                def zero_body(i, carry, row=row):
                    for u in range(8):
                        buf[row, pl.ds((i * 8 + u) * 16, 16)] = zv
                    return carry

                lax.fori_loop(0, _NCHUNK // (16 * 8), zero_body, 0)

        col_base = jnp.bitwise_and(lane8, 1) * _NSUPP

        def scatter(buf, r):
            row_idx = (r * _CLS_PER_ROUND + lane8) >> 1
            for j in range(_NSUPP):
                col_j = col_base + j
                cm_j = plsc.load_gather(cm_v, [row_idx, col_j], mask=lmask)
                w_j = plsc.load_gather(w_v, [row_idx, col_j], mask=lmask)
                plsc.addupdate_scatter(buf, [lane8, cm_j], w_j, mask=lmask)

        def unscatter(buf, r_prev):
            row_idx = (r_prev * _CLS_PER_ROUND + lane8) >> 1
            for j in range(_NSUPP):
                col_j = col_base + j
                cm_j = plsc.load_gather(cm_v, [row_idx, col_j], mask=lmask)
                plsc.store_scatter(buf, [lane8, cm_j], zv, mask=lmask)

        zero(buf0)
        in_cm.wait()
        in_w.wait()
        for r in range(_ROUNDS):
            b = r & 1
            buf = bufs[b]
            if r == 1:
                zero(buf1)
            elif r >= 2:
                copies[b].wait()
                unscatter(buf, r - 2)
            scatter(buf, r)
            row0 = pl.multiple_of(base_cls + r * _CLS_PER_ROUND,
                                  _CLS_PER_ROUND)
            copies[b] = pltpu.async_copy(
                buf, wt_hbm.at[pl.ds(row0, _CLS_PER_ROUND), :], sems[b])
        copies[0].wait()
        copies[1].wait()

    return k(cm2, w2)


_BN = 256


def _tc_matmul_exp_t(xb, wt):
    def body(wt_ref, x_ref, o_ref):
        acc = lax.dot_general(
            wt_ref[...].astype(jnp.bfloat16), x_ref[...],
            (((1,), (1,)), ((), ())),
            preferred_element_type=jnp.float32)
        o_ref[...] = jnp.exp(acc)

    return pl.pallas_call(
        body,
        grid=(_NCLS_PAD // _BN,),
        in_specs=[
            pl.BlockSpec((_BN, _NCHUNK), lambda j: (j, 0)),
            pl.BlockSpec((_B, _NCHUNK), lambda j: (0, 0)),
        ],
        out_specs=pl.BlockSpec((_BN, _B), lambda j: (j, 0)),
        out_shape=jax.ShapeDtypeStruct((_NCLASS, _B), jnp.float32),
    )(wt, xb)


def kernel(x, wSupp, chunk_map):
    pad = ((0, _NCLS_PAD - _NCLASS), (0, 0))
    cm2 = jnp.pad(chunk_map, pad).reshape(_ROWS2, 2 * _NSUPP)
    w2 = jnp.pad(wSupp, pad).reshape(_ROWS2, 2 * _NSUPP)
    wt = _sc_build_w(cm2, w2)
    xb = x.astype(jnp.bfloat16)
    return _tc_matmul_exp_t(xb, wt).T
